# baseline (speedup 1.0000x reference)
import functools

import jax
import jax.numpy as jnp
from jax import lax
from jax.experimental import pallas as pl
from jax.experimental.pallas import tpu as pltpu
from jax.experimental.pallas import tpu_sc as plsc

N_CAT_F = 26
F_PAD = 32
EMB = 16
VROWS = 100001
L = 16


def _sc_gather(tab_flat, cat_pad, num_pad, nodes, batch):
    info = plsc.get_sparse_core_info()
    nw = info.num_cores * info.num_subcores
    bpw = batch // nw
    n_idx_chunks = bpw // 128
    grp = 32
    n_groups = bpw // grp
    rows_per_grp = grp * F_PAD

    mesh = plsc.VectorSubcoreMesh(core_axis_name="c", subcore_axis_name="s")

    @functools.partial(
        pl.kernel,
        mesh=mesh,
        compiler_params=pltpu.CompilerParams(use_tc_tiling_on_sc=False),
        out_type=[
            jax.ShapeDtypeStruct((batch * F_PAD, EMB), jnp.float32),
            jax.ShapeDtypeStruct((batch, 16), jnp.float32),
        ],
        scratch_types=[
            pltpu.VMEM((bpw,), jnp.int32),
            pltpu.VMEM((bpw, F_PAD), jnp.int32),
            pltpu.VMEM((bpw * F_PAD,), jnp.int32),
            pltpu.VMEM((rows_per_grp, EMB), jnp.float32),
            pltpu.VMEM((bpw, 16), jnp.float32),
            pltpu.SemaphoreType.DMA,
            pltpu.SemaphoreType.DMA,
        ],
    )
    def k(tab_hbm, cat_hbm, num_hbm, nodes_hbm, emb_hbm, gnum_hbm,
          nodes_vm, gidx_vm, flat_vm, rows_vm, gnum_vm, sem_a, sem_b):
        wid = lax.axis_index("s") * info.num_cores + lax.axis_index("c")
        base = wid * bpw

        pltpu.sync_copy(nodes_hbm.at[pl.ds(base, bpw)], nodes_vm)

        hs = []
        for c in range(n_idx_chunks):
            idx = nodes_vm.at[pl.ds(c * 128, 128)]
            hs.append(pltpu.async_copy(
                cat_hbm.at[idx], gidx_vm.at[pl.ds(c * 128, 128)], sem_a))
            hs.append(pltpu.async_copy(
                num_hbm.at[idx], gnum_vm.at[pl.ds(c * 128, 128)], sem_b))
        for h in hs:
            h.wait()

        off0 = lax.iota(jnp.int32, 16) * VROWS
        off1 = jnp.minimum((lax.iota(jnp.int32, 16) + 16) * VROWS,
                           (N_CAT_F - 1) * VROWS)

        def fbody(i, carry):
            flat_vm[pl.ds(i * F_PAD, 16)] = gidx_vm[i, pl.ds(0, 16)] + off0
            flat_vm[pl.ds(i * F_PAD + 16, 16)] = gidx_vm[i, pl.ds(16, 16)] + off1
            return carry

        lax.fori_loop(0, bpw, fbody, 0)

        def gbody(g, carry):
            ghs = []
            for j in range(rows_per_grp // 128):
                ghs.append(pltpu.async_copy(
                    tab_hbm.at[flat_vm.at[pl.ds(g * rows_per_grp + j * 128, 128)]],
                    rows_vm.at[pl.ds(j * 128, 128)], sem_a))
            for h in ghs:
                h.wait()
            pltpu.sync_copy(
                rows_vm,
                emb_hbm.at[pl.ds((base + g * grp) * F_PAD, rows_per_grp)])
            return carry

        lax.fori_loop(0, n_groups, gbody, 0)

        pltpu.sync_copy(gnum_vm, gnum_hbm.at[pl.ds(base, bpw)])

    return k(tab_flat, cat_pad, num_pad, nodes)


def _tc_matmul(emb2d, gnum, wc_t, wn_t, batch):
    blk = 512
    k_dim = emb2d.shape[1]

    def body(emb_ref, gnum_ref, wc_ref, wn_ref, out_ref):
        out_ref[...] = (
            jnp.dot(emb_ref[...], wc_ref[...],
                    preferred_element_type=jnp.float32,
                    precision=lax.Precision.HIGHEST)
            + jnp.dot(gnum_ref[...], wn_ref[...],
                      preferred_element_type=jnp.float32,
                      precision=lax.Precision.HIGHEST))

    return pl.pallas_call(
        body,
        grid=(batch // blk,),
        in_specs=[
            pl.BlockSpec((blk, k_dim), lambda i: (i, 0)),
            pl.BlockSpec((blk, 16), lambda i: (i, 0)),
            pl.BlockSpec((k_dim, 64), lambda i: (0, 0)),
            pl.BlockSpec((16, 64), lambda i: (0, 0)),
        ],
        out_specs=pl.BlockSpec((blk, 64), lambda i: (i, 0)),
        out_shape=jax.ShapeDtypeStruct((batch, 64), jnp.float32),
    )(emb2d, gnum, wc_t, wn_t)


def kernel(tables, num_feats, W, b, cat_idx, nodes_v):
    n_cat, vrows, emb = tables.shape
    n_rows, n_num = num_feats.shape
    batch = nodes_v.shape[0]
    in_cat = n_cat * emb

    tab_flat = tables.reshape(n_cat * vrows, emb)
    cat_pad = jnp.pad(cat_idx.astype(jnp.int32), ((0, 0), (0, F_PAD - n_cat)))
    num_pad = jnp.concatenate(
        [num_feats,
         jnp.zeros((n_rows, 2), jnp.float32),
         jnp.ones((n_rows, 1), jnp.float32)], axis=1)

    wc_t = jnp.zeros((F_PAD * emb, 64), jnp.float32)
    wc_t = wc_t.at[:in_cat, :].set(W[:, :in_cat].T)
    wn_t = jnp.concatenate(
        [W[:, in_cat:].T, jnp.zeros((2, 64), jnp.float32), b[None, :]], axis=0)

    emb_rows, gnum = _sc_gather(tab_flat, cat_pad, num_pad,
                                nodes_v.astype(jnp.int32), batch)
    emb2d = emb_rows.reshape(batch, F_PAD * emb)
    return _tc_matmul(emb2d, gnum, wc_t, wn_t, batch)


# speedup vs baseline: 13.9702x; 13.9702x over previous
import functools

import jax
import jax.numpy as jnp
from jax import lax
from jax.experimental import pallas as pl
from jax.experimental.pallas import tpu as pltpu
from jax.experimental.pallas import tpu_sc as plsc

N_CAT = 26
EMB = 16
PADV = 100096
NROW = 100000
NLANE = N_CAT * EMB
Q = 4096


def _sc_stage_gather(t1d, cat1d, num1d, nodes, batch, n_num):
    nq = batch // Q
    out_rows = NLANE + n_num

    mesh = plsc.VectorSubcoreMesh(core_axis_name="c", subcore_axis_name="s")

    @functools.partial(
        pl.kernel,
        mesh=mesh,
        compiler_params=pltpu.CompilerParams(use_tc_tiling_on_sc=False,
                                             needs_layout_passes=False),
        out_type=jax.ShapeDtypeStruct((out_rows * batch,), jnp.float32),
        scratch_types=[
            pltpu.VMEM((PADV,), jnp.int32),
            pltpu.VMEM((Q,), jnp.int32),
            pltpu.VMEM((batch,), jnp.int32),
            pltpu.VMEM((Q,), jnp.float32),
        ],
    )
    def k(t_hbm, cat_hbm, num_hbm, nodes_hbm, out_hbm,
          big_vm, nodes_vm, cidx_vm, vals_vm):
        wid = lax.axis_index("s") * 2 + lax.axis_index("c")

        def gather_quarter(dst_vm, dst_base, _):
            def body(i, carry):
                idx16 = dst_vm[pl.ds(i * 16, 16)]
                return carry, idx16
            def g(i, carry):
                i16 = nodes_vm[pl.ds(i * 16, 16)]
                cidx_vm[pl.ds(dst_base + i * 16, 16)] = plsc.load_gather(
                    big_vm, [i16])
                return carry
            lax.fori_loop(0, Q // 16, g, 0)
            return None

        def phase1_field(f):
            pltpu.sync_copy(cat_hbm.at[pl.ds(f * NROW, NROW)],
                            big_vm.at[pl.ds(0, NROW)])
            for q in range(nq):
                pltpu.sync_copy(nodes_hbm.at[pl.ds(q * Q, Q)], nodes_vm)
                gather_quarter(nodes_vm, q * Q, None)

        def phase2_lane(src_ref, src_off, out_row):
            pltpu.sync_copy(src_ref.at[pl.ds(src_off, PADV)], big_vm)
            for q in range(nq):
                def g(i, carry):
                    i16 = cidx_vm[pl.ds(q * Q + i * 16, 16)]
                    v16 = plsc.load_gather(big_vm, [i16])
                    vals_vm[pl.ds(i * 16, 16)] = plsc.bitcast(v16, jnp.float32)
                    return carry
                lax.fori_loop(0, Q // 16, g, 0)
                pltpu.sync_copy(
                    vals_vm,
                    out_hbm.at[pl.ds(out_row * batch + q * Q, Q)])

        @pl.when(wid < N_CAT)
        def _field_worker():
            phase1_field(wid)

            def lane_body(e, carry):
                phase2_lane(t_hbm, (wid * EMB + e) * PADV,
                            wid * EMB + e)
                return carry
            lax.fori_loop(0, EMB, lane_body, 0)

        @pl.when(wid >= N_CAT)
        def _num_worker():
            for q in range(nq):
                pltpu.sync_copy(nodes_hbm.at[pl.ds(q * Q, Q)], nodes_vm)
                def cp(i, carry):
                    cidx_vm[pl.ds(q * Q + i * 16, 16)] = \
                        nodes_vm[pl.ds(i * 16, 16)]
                    return carry
                lax.fori_loop(0, Q // 16, cp, 0)

            def nlane_body(t, carry):
                kk = (wid - N_CAT) + 6 * t

                @pl.when(kk < n_num)
                def _():
                    phase2_lane(num_hbm, kk * NROW, NLANE + kk)
                return carry
            lax.fori_loop(0, 3, nlane_body, 0)

    return k(t1d, cat1d, num1d, nodes)


def _tc_repack(t_t, n_cat, emb):
    def body(in_ref, out_ref):
        for j in range(8):
            out_ref[pl.ds(j * PADV, PADV)] = in_ref[0, j, :]

    return pl.pallas_call(
        body,
        grid=(n_cat, emb // 8),
        in_specs=[pl.BlockSpec((1, 8, PADV), lambda f, h: (f, h, 0))],
        out_specs=pl.BlockSpec((8 * PADV,), lambda f, h: (f * (emb // 8) + h,)),
        out_shape=jax.ShapeDtypeStruct((n_cat * emb * PADV,), jnp.float32),
    )(t_t)


def _tc_matmul(emb_t, w_t, b2d, batch):
    blk = 2048
    kd = emb_t.shape[0]

    def body(emb_ref, w_ref, b_ref, out_ref):
        acc = jax.lax.dot_general(
            emb_ref[...], w_ref[...],
            dimension_numbers=(((0,), (0,)), ((), ())),
            preferred_element_type=jnp.float32,
            precision=lax.Precision.HIGHEST)
        out_ref[...] = acc + b_ref[0:1, :]

    return pl.pallas_call(
        body,
        grid=(batch // blk,),
        in_specs=[
            pl.BlockSpec((kd, blk), lambda i: (0, i)),
            pl.BlockSpec((kd, 64), lambda i: (0, 0)),
            pl.BlockSpec((8, 64), lambda i: (0, 0)),
        ],
        out_specs=pl.BlockSpec((blk, 64), lambda i: (i, 0)),
        out_shape=jax.ShapeDtypeStruct((batch, 64), jnp.float32),
    )(emb_t, w_t, b2d)


def kernel(tables, num_feats, W, b, cat_idx, nodes_v):
    n_cat, vrows, emb = tables.shape
    n_rows, n_num = num_feats.shape
    batch = nodes_v.shape[0]

    t_t = jnp.transpose(tables, (0, 2, 1))
    t1d = lax.bitcast_convert_type(_tc_repack(t_t, n_cat, emb), jnp.int32)

    cat1d = jnp.transpose(cat_idx).reshape(-1).astype(jnp.int32)
    num1d = lax.bitcast_convert_type(
        jnp.pad(jnp.transpose(num_feats).reshape(-1), (0, 96)), jnp.int32)

    emb_flat = _sc_stage_gather(t1d, cat1d, num1d,
                                nodes_v.astype(jnp.int32), batch, n_num)
    emb_t = emb_flat.reshape(n_cat * emb + n_num, batch)

    w_t = jnp.transpose(W)
    b2d = jnp.broadcast_to(b[None, :], (8, 64))
    return _tc_matmul(emb_t, w_t, b2d, batch)


# speedup vs baseline: 17.1199x; 1.2255x over previous
import functools

import jax
import jax.numpy as jnp
from jax import lax
from jax.experimental import pallas as pl
from jax.experimental.pallas import tpu as pltpu
from jax.experimental.pallas import tpu_sc as plsc

N_CAT = 26
EMB = 16
PADV = 100096
NROW = 100000
NLANE = N_CAT * EMB
Q = 4096


def _sc_cidx(cat1d_f32, nodes, batch, n_cat):
    nq = batch // Q
    mesh = plsc.VectorSubcoreMesh(core_axis_name="c", subcore_axis_name="s")

    @functools.partial(
        pl.kernel,
        mesh=mesh,
        compiler_params=pltpu.CompilerParams(use_tc_tiling_on_sc=False,
                                             needs_layout_passes=False),
        out_type=jax.ShapeDtypeStruct((n_cat * batch,), jnp.int32),
        scratch_types=[
            pltpu.VMEM((PADV,), jnp.float32),
            pltpu.VMEM((Q,), jnp.int32),
            pltpu.VMEM((Q,), jnp.int32),
        ],
    )
    def k(cat_hbm, nodes_hbm, cidx_hbm, big_vm, nodes_vm, vals_vm):
        wid = lax.axis_index("s") * 2 + lax.axis_index("c")

        @pl.when(wid < n_cat)
        def _():
            pltpu.sync_copy(cat_hbm.at[pl.ds(wid * NROW, NROW)],
                            big_vm.at[pl.ds(0, NROW)])
            for q in range(nq):
                pltpu.sync_copy(nodes_hbm.at[pl.ds(q * Q, Q)], nodes_vm)

                def g(i, carry):
                    i16 = nodes_vm[pl.ds(i * 16, 16)]
                    c16 = plsc.load_gather(big_vm, [i16])
                    vals_vm[pl.ds(i * 16, 16)] = plsc.bitcast(c16, jnp.int32)
                    return carry
                lax.fori_loop(0, Q // 16, g, 0)
                pltpu.sync_copy(vals_vm,
                                cidx_hbm.at[pl.ds(wid * batch + q * Q, Q)])

    return k(cat1d_f32, nodes)


def _sc_values(t1d, num1d, cidx1d, nodes, batch, n_num):
    nq = batch // Q
    out_rows = NLANE + n_num
    mesh = plsc.VectorSubcoreMesh(core_axis_name="c", subcore_axis_name="s")

    @functools.partial(
        pl.kernel,
        mesh=mesh,
        compiler_params=pltpu.CompilerParams(use_tc_tiling_on_sc=False,
                                             needs_layout_passes=False),
        out_type=jax.ShapeDtypeStruct((out_rows * batch,), jnp.float32),
        scratch_types=[
            pltpu.VMEM((PADV,), jnp.float32),
            pltpu.VMEM((batch,), jnp.int32),
            pltpu.VMEM((Q,), jnp.float32),
        ],
    )
    def k(t_hbm, num_hbm, cidx_hbm, nodes_hbm, out_hbm,
          big_vm, cidx_vm, vals_vm):
        wid = lax.axis_index("s") * 2 + lax.axis_index("c")

        def quarters(out_row):
            for q in range(nq):
                def g(i, carry):
                    i16 = cidx_vm[pl.ds(q * Q + i * 16, 16)]
                    vals_vm[pl.ds(i * 16, 16)] = plsc.load_gather(
                        big_vm, [i16])
                    return carry
                lax.fori_loop(0, Q // 16, g, 0)
                pltpu.sync_copy(
                    vals_vm,
                    out_hbm.at[pl.ds(out_row * batch + q * Q, Q)])

        def tjob(t, carry):
            j = wid + 32 * t
            pltpu.sync_copy(cidx_hbm.at[pl.ds((j // 16) * batch, batch)],
                            cidx_vm)
            pltpu.sync_copy(t_hbm.at[pl.ds(j * PADV, PADV)], big_vm)
            quarters(j)
            return carry
        lax.fori_loop(0, NLANE // 32, tjob, 0)

        @pl.when(wid < n_num)
        def _num_tail():
            pltpu.sync_copy(nodes_hbm, cidx_vm)
            pltpu.sync_copy(num_hbm.at[pl.ds(wid * NROW, PADV)], big_vm)
            quarters(NLANE + wid)

    return k(t1d, num1d, cidx1d, nodes)


def _tc_repack(t_t, n_cat, emb):
    def body(in_ref, out_ref):
        for j in range(8):
            out_ref[pl.ds(j * PADV, PADV)] = in_ref[0, j, :]

    return pl.pallas_call(
        body,
        grid=(n_cat, emb // 8),
        in_specs=[pl.BlockSpec((1, 8, PADV), lambda f, h: (f, h, 0))],
        out_specs=pl.BlockSpec((8 * PADV,), lambda f, h: (f * (emb // 8) + h,)),
        out_shape=jax.ShapeDtypeStruct((n_cat * emb * PADV,), jnp.float32),
    )(t_t)


def _tc_matmul(emb_t, w_t, b2d, batch):
    blk = 2048
    kd = emb_t.shape[0]

    def body(emb_ref, w_ref, b_ref, out_ref):
        acc = jax.lax.dot_general(
            emb_ref[...], w_ref[...],
            dimension_numbers=(((0,), (0,)), ((), ())),
            preferred_element_type=jnp.float32,
            precision=lax.Precision.HIGHEST)
        out_ref[...] = acc + b_ref[0:1, :]

    return pl.pallas_call(
        body,
        grid=(batch // blk,),
        in_specs=[
            pl.BlockSpec((kd, blk), lambda i: (0, i)),
            pl.BlockSpec((kd, 64), lambda i: (0, 0)),
            pl.BlockSpec((8, 64), lambda i: (0, 0)),
        ],
        out_specs=pl.BlockSpec((blk, 64), lambda i: (i, 0)),
        out_shape=jax.ShapeDtypeStruct((batch, 64), jnp.float32),
    )(emb_t, w_t, b2d)


def kernel(tables, num_feats, W, b, cat_idx, nodes_v):
    n_cat, vrows, emb = tables.shape
    n_rows, n_num = num_feats.shape
    batch = nodes_v.shape[0]
    nodes = nodes_v.astype(jnp.int32)

    t_t = jnp.transpose(tables, (0, 2, 1))
    t1d = _tc_repack(t_t, n_cat, emb)

    cat1d = lax.bitcast_convert_type(
        jnp.transpose(cat_idx).reshape(-1).astype(jnp.int32), jnp.float32)
    num1d = jnp.pad(jnp.transpose(num_feats).reshape(-1), (0, 96))

    cidx1d = _sc_cidx(cat1d, nodes, batch, n_cat)
    emb_flat = _sc_values(t1d, num1d, cidx1d, nodes, batch, n_num)
    emb_t = emb_flat.reshape(n_cat * emb + n_num, batch)

    w_t = jnp.transpose(W)
    b2d = jnp.broadcast_to(b[None, :], (8, 64))
    return _tc_matmul(emb_t, w_t, b2d, batch)


# speedup vs baseline: 19.0232x; 1.1112x over previous
import functools

import jax
import jax.numpy as jnp
from jax import lax
from jax.experimental import pallas as pl
from jax.experimental.pallas import tpu as pltpu
from jax.experimental.pallas import tpu_sc as plsc

N_CAT = 26
EMB = 16
PADV = 100096
NROW = 100000
NLANE = N_CAT * EMB
SPLIT = 216
Q = 4096
NW = 32

_SC_PARAMS = dict(
    compiler_params=pltpu.CompilerParams(use_tc_tiling_on_sc=False,
                                         needs_layout_passes=False),
    mesh=plsc.VectorSubcoreMesh(core_axis_name="c", subcore_axis_name="s"),
)


def _sc_cidx(cat1d, nodes, batch, n_cat):
    nq = batch // Q

    @functools.partial(
        pl.kernel,
        out_type=jax.ShapeDtypeStruct((n_cat * batch,), jnp.int32),
        scratch_types=[
            pltpu.VMEM((PADV,), jnp.int32),
            pltpu.VMEM((Q,), jnp.int32),
            pltpu.VMEM((Q,), jnp.int32),
        ],
        **_SC_PARAMS,
    )
    def k(cat_hbm, nodes_hbm, cidx_hbm, big_vm, nodes_vm, vals_vm):
        wid = lax.axis_index("s") * 2 + lax.axis_index("c")

        @pl.when(wid < n_cat)
        def _():
            pltpu.sync_copy(cat_hbm.at[pl.ds(wid * NROW, NROW)],
                            big_vm.at[pl.ds(0, NROW)])
            for q in range(nq):
                pltpu.sync_copy(nodes_hbm.at[pl.ds(q * Q, Q)], nodes_vm)

                def g(i, carry):
                    i16 = nodes_vm[pl.ds(i * 16, 16)]
                    vals_vm[pl.ds(i * 16, 16)] = plsc.load_gather(
                        big_vm, [i16])
                    return carry
                lax.fori_loop(0, Q // 16, g, 0)
                pltpu.sync_copy(vals_vm,
                                cidx_hbm.at[pl.ds(wid * batch + q * Q, Q)])

    return k(cat1d, nodes)


def _sc_values(t1d, cidx1d, batch, lane_lo, lane_hi, num1d=None, nodes=None,
               n_num=0):
    nq = batch // Q
    n_tab = lane_hi - lane_lo
    out_rows = n_tab + n_num
    extra = (num1d, nodes) if n_num else ()

    @functools.partial(
        pl.kernel,
        out_type=jax.ShapeDtypeStruct((out_rows * batch,), jnp.float32),
        scratch_types=[
            pltpu.VMEM((PADV,), jnp.float32),
            pltpu.VMEM((batch,), jnp.int32),
            pltpu.VMEM((Q,), jnp.float32),
        ],
        **_SC_PARAMS,
    )
    def k(t_hbm, cidx_hbm, *rest):
        (num_hbm, nodes_hbm) = rest[:2] if n_num else (None, None)
        out_hbm, big_vm, cidx_vm, vals_vm = rest[-4:]
        wid = lax.axis_index("s") * 2 + lax.axis_index("c")

        def quarters(out_row):
            for q in range(nq):
                def g(i, carry):
                    i16 = cidx_vm[pl.ds(q * Q + i * 16, 16)]
                    vals_vm[pl.ds(i * 16, 16)] = plsc.load_gather(
                        big_vm, [i16])
                    return carry
                lax.fori_loop(0, Q // 16, g, 0)
                pltpu.sync_copy(
                    vals_vm,
                    out_hbm.at[pl.ds(out_row * batch + q * Q, Q)])

        def tjob(t, carry):
            j = wid + NW * t

            @pl.when(j < n_tab)
            def _():
                f = (lane_lo + j) // EMB
                pltpu.sync_copy(cidx_hbm.at[pl.ds(f * batch, batch)], cidx_vm)
                pltpu.sync_copy(t_hbm.at[pl.ds(j * PADV, PADV)], big_vm)
                quarters(j)
            return carry
        lax.fori_loop(0, (n_tab + NW - 1) // NW, tjob, 0)

        if n_num:
            @pl.when(wid < n_num)
            def _num_tail():
                pltpu.sync_copy(nodes_hbm, cidx_vm)
                pltpu.sync_copy(num_hbm.at[pl.ds(wid * NROW, PADV)], big_vm)
                quarters(n_tab + wid)

    return k(t1d, cidx1d, *extra)


def _tc_repack(t_t, blk_lo, n_blk):
    def body(in_ref, out_ref):
        for j in range(8):
            out_ref[pl.ds(j * PADV, PADV)] = in_ref[0, j, :]

    return pl.pallas_call(
        body,
        grid=(n_blk,),
        in_specs=[pl.BlockSpec(
            (1, 8, PADV), lambda b: ((b + blk_lo) // 2, (b + blk_lo) % 2, 0))],
        out_specs=pl.BlockSpec((8 * PADV,), lambda b: (b,)),
        out_shape=jax.ShapeDtypeStruct((n_blk * 8 * PADV,), jnp.float32),
    )(t_t)


def _tc_matmul(emb1, emb2, w1, w2, b2d, batch):
    blk = 2048
    k1, k2 = emb1.shape[0], emb2.shape[0]

    def body(e1_ref, e2_ref, w1_ref, w2_ref, b_ref, out_ref):
        dn = (((0,), (0,)), ((), ()))
        acc = jax.lax.dot_general(
            e1_ref[...], w1_ref[...], dimension_numbers=dn,
            preferred_element_type=jnp.float32,
            precision=lax.Precision.HIGHEST)
        acc += jax.lax.dot_general(
            e2_ref[...], w2_ref[...], dimension_numbers=dn,
            preferred_element_type=jnp.float32,
            precision=lax.Precision.HIGHEST)
        out_ref[...] = acc + b_ref[0:1, :]

    return pl.pallas_call(
        body,
        grid=(batch // blk,),
        in_specs=[
            pl.BlockSpec((k1, blk), lambda i: (0, i)),
            pl.BlockSpec((k2, blk), lambda i: (0, i)),
            pl.BlockSpec((k1, 64), lambda i: (0, 0)),
            pl.BlockSpec((k2, 64), lambda i: (0, 0)),
            pl.BlockSpec((8, 64), lambda i: (0, 0)),
        ],
        out_specs=pl.BlockSpec((blk, 64), lambda i: (i, 0)),
        out_shape=jax.ShapeDtypeStruct((batch, 64), jnp.float32),
    )(emb1, emb2, w1, w2, b2d)


def kernel(tables, num_feats, W, b, cat_idx, nodes_v):
    n_cat, vrows, emb = tables.shape
    n_rows, n_num = num_feats.shape
    batch = nodes_v.shape[0]
    nodes = nodes_v.astype(jnp.int32)

    t_t = jnp.transpose(tables, (0, 2, 1))
    cat1d = jnp.transpose(cat_idx).reshape(-1).astype(jnp.int32)
    num1d = jnp.pad(jnp.transpose(num_feats).reshape(-1), (0, 96))

    nb1 = SPLIT // 8
    tA = _tc_repack(t_t, 0, nb1)
    tB = _tc_repack(t_t, nb1, NLANE // 8 - nb1)

    cidx1d = _sc_cidx(cat1d, nodes, batch, n_cat)
    emb1 = _sc_values(tA, cidx1d, batch, 0, SPLIT)
    emb2 = _sc_values(tB, cidx1d, batch, SPLIT, NLANE,
                      num1d=num1d, nodes=nodes, n_num=n_num)

    w_t = jnp.transpose(W)
    w1 = w_t[:SPLIT]
    w2 = w_t[SPLIT:]
    b2d = jnp.broadcast_to(b[None, :], (8, 64))
    return _tc_matmul(emb1.reshape(SPLIT, batch),
                      emb2.reshape(NLANE - SPLIT + n_num, batch),
                      w1, w2, b2d, batch)


# speedup vs baseline: 19.3475x; 1.0170x over previous
import functools

import jax
import jax.numpy as jnp
from jax import lax
from jax.experimental import pallas as pl
from jax.experimental.pallas import tpu as pltpu
from jax.experimental.pallas import tpu_sc as plsc

N_CAT = 26
EMB = 16
PADV = 100096
NROW = 100000
NLANE = N_CAT * EMB
SPLIT = 216
Q = 4096
NW = 32

_SC_PARAMS = dict(
    compiler_params=pltpu.CompilerParams(use_tc_tiling_on_sc=False,
                                         needs_layout_passes=False),
    mesh=plsc.VectorSubcoreMesh(core_axis_name="c", subcore_axis_name="s"),
)


def _sc_cidx(cat1d, nodes, batch, n_cat):
    nq = batch // Q

    @functools.partial(
        pl.kernel,
        out_type=jax.ShapeDtypeStruct((n_cat * batch,), jnp.int32),
        scratch_types=[
            pltpu.VMEM((PADV,), jnp.int32),
            pltpu.VMEM((Q,), jnp.int32),
            pltpu.VMEM((Q,), jnp.int32),
        ],
        **_SC_PARAMS,
    )
    def k(cat_hbm, nodes_hbm, cidx_hbm, big_vm, nodes_vm, vals_vm):
        wid = lax.axis_index("s") * 2 + lax.axis_index("c")

        @pl.when(wid < n_cat)
        def _():
            pltpu.sync_copy(cat_hbm.at[pl.ds(wid * NROW, NROW)],
                            big_vm.at[pl.ds(0, NROW)])
            for q in range(nq):
                pltpu.sync_copy(nodes_hbm.at[pl.ds(q * Q, Q)], nodes_vm)

                def g(i, carry):
                    i16 = nodes_vm[pl.ds(i * 16, 16)]
                    vals_vm[pl.ds(i * 16, 16)] = plsc.load_gather(
                        big_vm, [i16])
                    return carry
                lax.fori_loop(0, Q // 16, g, 0)
                pltpu.sync_copy(vals_vm,
                                cidx_hbm.at[pl.ds(wid * batch + q * Q, Q)])

    return k(cat1d, nodes)


def _sc_values(t1d, cidx1d, batch, lane_lo, lane_hi, num1d=None, nodes=None,
               n_num=0):
    nq = batch // Q
    n_tab = lane_hi - lane_lo
    out_rows = n_tab + n_num
    extra = (num1d, nodes) if n_num else ()

    @functools.partial(
        pl.kernel,
        out_type=jax.ShapeDtypeStruct((out_rows * batch,), jnp.float32),
        scratch_types=[
            pltpu.VMEM((PADV,), jnp.float32),
            pltpu.VMEM((batch,), jnp.int32),
            pltpu.VMEM((Q,), jnp.float32),
        ],
        **_SC_PARAMS,
    )
    def k(t_hbm, cidx_hbm, *rest):
        (num_hbm, nodes_hbm) = rest[:2] if n_num else (None, None)
        out_hbm, big_vm, cidx_vm, vals_vm = rest[-4:]
        wid = lax.axis_index("s") * 2 + lax.axis_index("c")

        def quarters(out_row):
            for q in range(nq):
                def g(i, carry):
                    i16 = cidx_vm[pl.ds(q * Q + i * 16, 16)]
                    vals_vm[pl.ds(i * 16, 16)] = plsc.load_gather(
                        big_vm, [i16])
                    return carry
                lax.fori_loop(0, Q // 16, g, 0)
                pltpu.sync_copy(
                    vals_vm,
                    out_hbm.at[pl.ds(out_row * batch + q * Q, Q)])

        def tjob(t, carry):
            j = wid + NW * t

            @pl.when(j < n_tab)
            def _():
                f = (lane_lo + j) // EMB
                pltpu.sync_copy(cidx_hbm.at[pl.ds(f * batch, batch)], cidx_vm)
                pltpu.sync_copy(t_hbm.at[pl.ds(j * PADV, PADV)], big_vm)
                quarters(j)
            return carry
        lax.fori_loop(0, (n_tab + NW - 1) // NW, tjob, 0)

        if n_num:
            kk = wid - (NW - n_num)

            @pl.when(kk >= 0)
            def _num_tail():
                pltpu.sync_copy(nodes_hbm, cidx_vm)
                pltpu.sync_copy(num_hbm.at[pl.ds(kk * NROW, PADV)], big_vm)
                quarters(n_tab + kk)

    return k(t1d, cidx1d, *extra)


def _tc_repack(t_t, blk_lo, n_blk):
    def body(in_ref, out_ref):
        for j in range(8):
            out_ref[pl.ds(j * PADV, PADV)] = in_ref[0, j, :]

    return pl.pallas_call(
        body,
        grid=(n_blk,),
        in_specs=[pl.BlockSpec(
            (1, 8, PADV), lambda b: ((b + blk_lo) // 2, (b + blk_lo) % 2, 0))],
        out_specs=pl.BlockSpec((8 * PADV,), lambda b: (b,)),
        out_shape=jax.ShapeDtypeStruct((n_blk * 8 * PADV,), jnp.float32),
    )(t_t)


def _tc_matmul(emb, w, b2d, batch, acc_in=None):
    blk = 2048
    kd = emb.shape[0]
    extra = (acc_in,) if acc_in is not None else ()

    def body(emb_ref, w_ref, b_ref, *rest):
        acc_ref = rest[0] if acc_in is not None else None
        out_ref = rest[-1]
        acc = jax.lax.dot_general(
            emb_ref[...], w_ref[...],
            dimension_numbers=(((0,), (0,)), ((), ())),
            preferred_element_type=jnp.float32,
            precision=lax.Precision.HIGHEST)
        acc += b_ref[0:1, :] if acc_in is None else acc_ref[...]
        out_ref[...] = acc

    return pl.pallas_call(
        body,
        grid=(batch // blk,),
        in_specs=[
            pl.BlockSpec((kd, blk), lambda i: (0, i)),
            pl.BlockSpec((kd, 64), lambda i: (0, 0)),
            pl.BlockSpec((8, 64), lambda i: (0, 0)),
        ] + ([pl.BlockSpec((blk, 64), lambda i: (i, 0))] if extra else []),
        out_specs=pl.BlockSpec((blk, 64), lambda i: (i, 0)),
        out_shape=jax.ShapeDtypeStruct((batch, 64), jnp.float32),
    )(emb, w, b2d, *extra)


def kernel(tables, num_feats, W, b, cat_idx, nodes_v):
    n_cat, vrows, emb = tables.shape
    n_rows, n_num = num_feats.shape
    batch = nodes_v.shape[0]
    nodes = nodes_v.astype(jnp.int32)

    t_t = jnp.transpose(tables, (0, 2, 1))
    cat1d = jnp.transpose(cat_idx).reshape(-1).astype(jnp.int32)
    num1d = jnp.pad(jnp.transpose(num_feats).reshape(-1), (0, 96))

    nb1 = SPLIT // 8
    tA = _tc_repack(t_t, 0, nb1)
    tB = _tc_repack(t_t, nb1, NLANE // 8 - nb1)

    cidx1d = _sc_cidx(cat1d, nodes, batch, n_cat)
    emb1 = _sc_values(tA, cidx1d, batch, 0, SPLIT)
    emb2 = _sc_values(tB, cidx1d, batch, SPLIT, NLANE,
                      num1d=num1d, nodes=nodes, n_num=n_num)

    w_t = jnp.transpose(W)
    b2d = jnp.broadcast_to(b[None, :], (8, 64))
    part1 = _tc_matmul(emb1.reshape(SPLIT, batch), w_t[:SPLIT], b2d, batch)
    return _tc_matmul(emb2.reshape(NLANE - SPLIT + n_num, batch),
                      w_t[SPLIT:], b2d, batch, acc_in=part1)


# speedup vs baseline: 20.5317x; 1.0612x over previous
import functools

import jax
import jax.numpy as jnp
from jax import lax
from jax.experimental import pallas as pl
from jax.experimental.pallas import tpu as pltpu
from jax.experimental.pallas import tpu_sc as plsc

N_CAT = 26
EMB = 16
PADV = 100096
NROW = 100000
NLANE = N_CAT * EMB
SPLIT = 216
Q = 4096
NW = 32

_SC_PARAMS = dict(
    compiler_params=pltpu.CompilerParams(use_tc_tiling_on_sc=False,
                                         needs_layout_passes=False),
    mesh=plsc.VectorSubcoreMesh(core_axis_name="c", subcore_axis_name="s"),
)


def _sc_cidx(cat1d, nodes, batch, n_cat):
    nq = batch // Q

    @functools.partial(
        pl.kernel,
        out_type=jax.ShapeDtypeStruct((n_cat * batch,), jnp.int32),
        scratch_types=[
            pltpu.VMEM((PADV,), jnp.int32),
            pltpu.VMEM((Q,), jnp.int32),
            pltpu.VMEM((Q,), jnp.int32),
        ],
        **_SC_PARAMS,
    )
    def k(cat_hbm, nodes_hbm, cidx_hbm, big_vm, nodes_vm, vals_vm):
        wid = lax.axis_index("s") * 2 + lax.axis_index("c")

        @pl.when(wid < n_cat)
        def _():
            pltpu.sync_copy(cat_hbm.at[pl.ds(wid * NROW, NROW)],
                            big_vm.at[pl.ds(0, NROW)])
            for q in range(nq):
                pltpu.sync_copy(nodes_hbm.at[pl.ds(q * Q, Q)], nodes_vm)

                def g(i, carry):
                    for u in range(4):
                        i16 = nodes_vm[pl.ds(i * 64 + u * 16, 16)]
                        vals_vm[pl.ds(i * 64 + u * 16, 16)] = \
                            plsc.load_gather(big_vm, [i16])
                    return carry
                lax.fori_loop(0, Q // 64, g, 0)
                pltpu.sync_copy(vals_vm,
                                cidx_hbm.at[pl.ds(wid * batch + q * Q, Q)])

    return k(cat1d, nodes)


def _sc_values(t1d, cidx1d, batch, lane_lo, lane_hi, num1d=None, nodes=None,
               n_num=0):
    nq = batch // Q
    n_tab = lane_hi - lane_lo
    out_rows = n_tab + n_num
    extra = (num1d, nodes) if n_num else ()

    @functools.partial(
        pl.kernel,
        out_type=jax.ShapeDtypeStruct((out_rows * batch,), jnp.float32),
        scratch_types=[
            pltpu.VMEM((PADV,), jnp.float32),
            pltpu.VMEM((batch,), jnp.int32),
            pltpu.VMEM((Q,), jnp.float32),
        ],
        **_SC_PARAMS,
    )
    def k(t_hbm, cidx_hbm, *rest):
        (num_hbm, nodes_hbm) = rest[:2] if n_num else (None, None)
        out_hbm, big_vm, cidx_vm, vals_vm = rest[-4:]
        wid = lax.axis_index("s") * 2 + lax.axis_index("c")

        def quarters(out_row):
            for q in range(nq):
                def g(i, carry):
                    for u in range(4):
                        i16 = cidx_vm[pl.ds(q * Q + i * 64 + u * 16, 16)]
                        vals_vm[pl.ds(i * 64 + u * 16, 16)] = \
                            plsc.load_gather(big_vm, [i16])
                    return carry
                lax.fori_loop(0, Q // 64, g, 0)
                pltpu.sync_copy(
                    vals_vm,
                    out_hbm.at[pl.ds(out_row * batch + q * Q, Q)])

        def tjob(t, carry):
            j = wid + NW * t

            @pl.when(j < n_tab)
            def _():
                f = (lane_lo + j) // EMB
                pltpu.sync_copy(cidx_hbm.at[pl.ds(f * batch, batch)], cidx_vm)
                pltpu.sync_copy(t_hbm.at[pl.ds(j * PADV, PADV)], big_vm)
                quarters(j)
            return carry
        lax.fori_loop(0, (n_tab + NW - 1) // NW, tjob, 0)

        if n_num:
            kk = wid - (NW - n_num)

            @pl.when(kk >= 0)
            def _num_tail():
                pltpu.sync_copy(nodes_hbm, cidx_vm)
                pltpu.sync_copy(num_hbm.at[pl.ds(kk * NROW, PADV)], big_vm)
                quarters(n_tab + kk)

    return k(t1d, cidx1d, *extra)


def _tc_repack(t_t, blk_lo, n_blk):
    def body(in_ref, out_ref):
        for j in range(8):
            out_ref[pl.ds(j * PADV, PADV)] = in_ref[0, j, :]

    return pl.pallas_call(
        body,
        grid=(n_blk,),
        in_specs=[pl.BlockSpec(
            (1, 8, PADV), lambda b: ((b + blk_lo) // 2, (b + blk_lo) % 2, 0))],
        out_specs=pl.BlockSpec((8 * PADV,), lambda b: (b,)),
        out_shape=jax.ShapeDtypeStruct((n_blk * 8 * PADV,), jnp.float32),
    )(t_t)


def _tc_matmul(emb, w, b2d, batch, acc_in=None):
    blk = 2048
    kd = emb.shape[0]
    extra = (acc_in,) if acc_in is not None else ()

    def body(emb_ref, w_ref, b_ref, *rest):
        acc_ref = rest[0] if acc_in is not None else None
        out_ref = rest[-1]
        acc = jax.lax.dot_general(
            emb_ref[...], w_ref[...],
            dimension_numbers=(((0,), (0,)), ((), ())),
            preferred_element_type=jnp.float32,
            precision=lax.Precision.HIGHEST)
        acc += b_ref[0:1, :] if acc_in is None else acc_ref[...]
        out_ref[...] = acc

    return pl.pallas_call(
        body,
        grid=(batch // blk,),
        in_specs=[
            pl.BlockSpec((kd, blk), lambda i: (0, i)),
            pl.BlockSpec((kd, 64), lambda i: (0, 0)),
            pl.BlockSpec((8, 64), lambda i: (0, 0)),
        ] + ([pl.BlockSpec((blk, 64), lambda i: (i, 0))] if extra else []),
        out_specs=pl.BlockSpec((blk, 64), lambda i: (i, 0)),
        out_shape=jax.ShapeDtypeStruct((batch, 64), jnp.float32),
    )(emb, w, b2d, *extra)


def kernel(tables, num_feats, W, b, cat_idx, nodes_v):
    n_cat, vrows, emb = tables.shape
    n_rows, n_num = num_feats.shape
    batch = nodes_v.shape[0]
    nodes = nodes_v.astype(jnp.int32)

    t_t = jnp.transpose(tables, (0, 2, 1))
    cat1d = jnp.transpose(cat_idx).reshape(-1).astype(jnp.int32)
    num1d = jnp.pad(jnp.transpose(num_feats).reshape(-1), (0, 96))

    nb1 = SPLIT // 8
    tA = _tc_repack(t_t, 0, nb1)
    tB = _tc_repack(t_t, nb1, NLANE // 8 - nb1)

    cidx1d = _sc_cidx(cat1d, nodes, batch, n_cat)
    emb1 = _sc_values(tA, cidx1d, batch, 0, SPLIT)
    emb2 = _sc_values(tB, cidx1d, batch, SPLIT, NLANE,
                      num1d=num1d, nodes=nodes, n_num=n_num)

    w_t = jnp.transpose(W)
    b2d = jnp.broadcast_to(b[None, :], (8, 64))
    part1 = _tc_matmul(emb1.reshape(SPLIT, batch), w_t[:SPLIT], b2d, batch)
    return _tc_matmul(emb2.reshape(NLANE - SPLIT + n_num, batch),
                      w_t[SPLIT:], b2d, batch, acc_in=part1)


# speedup vs baseline: 20.5647x; 1.0016x over previous
import functools

import jax
import jax.numpy as jnp
from jax import lax
from jax.experimental import pallas as pl
from jax.experimental.pallas import tpu as pltpu
from jax.experimental.pallas import tpu_sc as plsc

N_CAT = 26
EMB = 16
PADV = 100096
NROW = 100000
NLANE = N_CAT * EMB
SPLIT = 216
Q = 4096
NW = 32

_SC_PARAMS = dict(
    compiler_params=pltpu.CompilerParams(use_tc_tiling_on_sc=False,
                                         needs_layout_passes=False),
    mesh=plsc.VectorSubcoreMesh(core_axis_name="c", subcore_axis_name="s"),
)


def _sc_cidx(cat1d, nodes, batch, n_cat):
    nq = batch // Q

    @functools.partial(
        pl.kernel,
        out_type=jax.ShapeDtypeStruct((n_cat * batch,), jnp.int32),
        scratch_types=[
            pltpu.VMEM((PADV,), jnp.int32),
            pltpu.VMEM((Q,), jnp.int32),
            pltpu.VMEM((Q,), jnp.int32),
        ],
        **_SC_PARAMS,
    )
    def k(cat_hbm, nodes_hbm, cidx_hbm, big_vm, nodes_vm, vals_vm):
        wid = lax.axis_index("s") * 2 + lax.axis_index("c")

        @pl.when(wid < n_cat)
        def _():
            pltpu.sync_copy(cat_hbm.at[pl.ds(wid * NROW, NROW)],
                            big_vm.at[pl.ds(0, NROW)])
            for q in range(nq):
                pltpu.sync_copy(nodes_hbm.at[pl.ds(q * Q, Q)], nodes_vm)

                def g(i, carry):
                    for u in range(8):
                        i16 = nodes_vm[pl.ds(i * 128 + u * 16, 16)]
                        vals_vm[pl.ds(i * 128 + u * 16, 16)] = \
                            plsc.load_gather(big_vm, [i16])
                    return carry
                lax.fori_loop(0, Q // 128, g, 0)
                pltpu.sync_copy(vals_vm,
                                cidx_hbm.at[pl.ds(wid * batch + q * Q, Q)])

    return k(cat1d, nodes)


def _sc_values(t1d, cidx1d, batch, lane_lo, lane_hi, num1d=None, nodes=None,
               n_num=0):
    nq = batch // Q
    n_tab = lane_hi - lane_lo
    out_rows = n_tab + n_num
    extra = (num1d, nodes) if n_num else ()

    @functools.partial(
        pl.kernel,
        out_type=jax.ShapeDtypeStruct((out_rows * batch,), jnp.float32),
        scratch_types=[
            pltpu.VMEM((PADV,), jnp.float32),
            pltpu.VMEM((batch,), jnp.int32),
            pltpu.VMEM((Q,), jnp.float32),
        ],
        **_SC_PARAMS,
    )
    def k(t_hbm, cidx_hbm, *rest):
        (num_hbm, nodes_hbm) = rest[:2] if n_num else (None, None)
        out_hbm, big_vm, cidx_vm, vals_vm = rest[-4:]
        wid = lax.axis_index("s") * 2 + lax.axis_index("c")

        def quarters(out_row):
            for q in range(nq):
                def g(i, carry):
                    for u in range(8):
                        i16 = cidx_vm[pl.ds(q * Q + i * 128 + u * 16, 16)]
                        vals_vm[pl.ds(i * 128 + u * 16, 16)] = \
                            plsc.load_gather(big_vm, [i16])
                    return carry
                lax.fori_loop(0, Q // 128, g, 0)
                pltpu.sync_copy(
                    vals_vm,
                    out_hbm.at[pl.ds(out_row * batch + q * Q, Q)])

        def tjob(t, carry):
            j = wid + NW * t

            @pl.when(j < n_tab)
            def _():
                f = (lane_lo + j) // EMB
                pltpu.sync_copy(cidx_hbm.at[pl.ds(f * batch, batch)], cidx_vm)
                pltpu.sync_copy(t_hbm.at[pl.ds(j * PADV, PADV)], big_vm)
                quarters(j)
            return carry
        lax.fori_loop(0, (n_tab + NW - 1) // NW, tjob, 0)

        if n_num:
            kk = wid - (NW - n_num)

            @pl.when(kk >= 0)
            def _num_tail():
                pltpu.sync_copy(nodes_hbm, cidx_vm)
                pltpu.sync_copy(num_hbm.at[pl.ds(kk * NROW, PADV)], big_vm)
                quarters(n_tab + kk)

    return k(t1d, cidx1d, *extra)


def _tc_repack(t_t, blk_lo, n_blk):
    def body(in_ref, out_ref):
        for j in range(8):
            out_ref[pl.ds(j * PADV, PADV)] = in_ref[0, j, :]

    return pl.pallas_call(
        body,
        grid=(n_blk,),
        in_specs=[pl.BlockSpec(
            (1, 8, PADV), lambda b: ((b + blk_lo) // 2, (b + blk_lo) % 2, 0))],
        out_specs=pl.BlockSpec((8 * PADV,), lambda b: (b,)),
        out_shape=jax.ShapeDtypeStruct((n_blk * 8 * PADV,), jnp.float32),
    )(t_t)


def _tc_matmul(emb, w, b2d, batch, acc_in=None):
    blk = 2048
    kd = emb.shape[0]
    extra = (acc_in,) if acc_in is not None else ()

    def body(emb_ref, w_ref, b_ref, *rest):
        acc_ref = rest[0] if acc_in is not None else None
        out_ref = rest[-1]
        acc = jax.lax.dot_general(
            emb_ref[...], w_ref[...],
            dimension_numbers=(((0,), (0,)), ((), ())),
            preferred_element_type=jnp.float32,
            precision=lax.Precision.HIGHEST)
        acc += b_ref[0:1, :] if acc_in is None else acc_ref[...]
        out_ref[...] = acc

    return pl.pallas_call(
        body,
        grid=(batch // blk,),
        in_specs=[
            pl.BlockSpec((kd, blk), lambda i: (0, i)),
            pl.BlockSpec((kd, 64), lambda i: (0, 0)),
            pl.BlockSpec((8, 64), lambda i: (0, 0)),
        ] + ([pl.BlockSpec((blk, 64), lambda i: (i, 0))] if extra else []),
        out_specs=pl.BlockSpec((blk, 64), lambda i: (i, 0)),
        out_shape=jax.ShapeDtypeStruct((batch, 64), jnp.float32),
    )(emb, w, b2d, *extra)


def kernel(tables, num_feats, W, b, cat_idx, nodes_v):
    n_cat, vrows, emb = tables.shape
    n_rows, n_num = num_feats.shape
    batch = nodes_v.shape[0]
    nodes = nodes_v.astype(jnp.int32)

    t_t = jnp.transpose(tables, (0, 2, 1))
    cat1d = jnp.transpose(cat_idx).reshape(-1).astype(jnp.int32)
    num1d = jnp.pad(jnp.transpose(num_feats).reshape(-1), (0, 96))

    nb1 = SPLIT // 8
    tA = _tc_repack(t_t, 0, nb1)
    tB = _tc_repack(t_t, nb1, NLANE // 8 - nb1)

    cidx1d = _sc_cidx(cat1d, nodes, batch, n_cat)
    emb1 = _sc_values(tA, cidx1d, batch, 0, SPLIT)
    emb2 = _sc_values(tB, cidx1d, batch, SPLIT, NLANE,
                      num1d=num1d, nodes=nodes, n_num=n_num)

    w_t = jnp.transpose(W)
    b2d = jnp.broadcast_to(b[None, :], (8, 64))
    part1 = _tc_matmul(emb1.reshape(SPLIT, batch), w_t[:SPLIT], b2d, batch)
    return _tc_matmul(emb2.reshape(NLANE - SPLIT + n_num, batch),
                      w_t[SPLIT:], b2d, batch, acc_in=part1)
